# Initial kernel scaffold; baseline (speedup 1.0000x reference)
import functools

import jax
import jax.numpy as jnp
from jax.experimental import pallas as pl
from jax.experimental.pallas import tpu as pltpu

B, NUM_NODE, HID, HEAD, RD = 4, 12500, 6, 8, 8
N = B * NUM_NODE
POOL = 100
NUM_CLASSES = 33
F = HEAD * HID
LIN_IN = (NUM_NODE // POOL) * F
K_BLK = 500
K_STEPS = LIN_IN // K_BLK


def _head_body(h_blk, w1_blk, lin1_b, lin2_W, lin2_b, cls_W, cls_b, out_ref, acc):
    k = pl.program_id(0)

    @pl.when(k == 0)
    def _():
        acc[...] = jnp.zeros_like(acc)

    acc[...] += jnp.dot(h_blk[...], w1_blk[...],
                        preferred_element_type=jnp.float32)

    @pl.when(k == K_STEPS - 1)
    def _():
        h1 = jnp.maximum(acc[...] + lin1_b[...], 0.0)
        h2 = jnp.maximum(
            jnp.dot(h1, lin2_W[...], preferred_element_type=jnp.float32)
            + lin2_b[...], 0.0)
        logits = jnp.dot(h2, cls_W[...],
                         preferred_element_type=jnp.float32) + cls_b[...]
        m = jnp.max(logits, axis=1, keepdims=True)
        z = logits - m
        lse = jnp.log(jnp.sum(jnp.exp(z), axis=1, keepdims=True))
        out_ref[...] = z - lse


def _dense_head(h, lin1_W, lin1_b, lin2_W, lin2_b, cls_W, cls_b):
    lin1_b = lin1_b.reshape(1, -1)
    lin2_b = lin2_b.reshape(1, -1)
    cls_b = cls_b.reshape(1, -1)
    return pl.pallas_call(
        _head_body,
        grid=(K_STEPS,),
        in_specs=[
            pl.BlockSpec((B, K_BLK), lambda k: (0, k)),
            pl.BlockSpec((K_BLK, LIN_IN // 4), lambda k: (k, 0)),
            pl.BlockSpec((1, LIN_IN // 4), lambda k: (0, 0)),
            pl.BlockSpec((LIN_IN // 4, 64), lambda k: (0, 0)),
            pl.BlockSpec((1, 64), lambda k: (0, 0)),
            pl.BlockSpec((64, NUM_CLASSES), lambda k: (0, 0)),
            pl.BlockSpec((1, NUM_CLASSES), lambda k: (0, 0)),
        ],
        out_specs=pl.BlockSpec((B, NUM_CLASSES), lambda k: (0, 0)),
        out_shape=jax.ShapeDtypeStruct((B, NUM_CLASSES), jnp.float32),
        scratch_shapes=[pltpu.VMEM((B, LIN_IN // 4), jnp.float32)],
    )(h, lin1_W, lin1_b, lin2_W, lin2_b, cls_W, cls_b)


def _add_self_loops(src, dst, eattr, n):
    ones = jnp.ones((src.shape[0],), dtype=eattr.dtype)
    cnt = jax.ops.segment_sum(ones, dst, num_segments=n)
    sums = jax.ops.segment_sum(eattr, dst, num_segments=n)
    loop_attr = sums / jnp.maximum(cnt, 1.0)[:, None]
    loop = jnp.arange(n, dtype=src.dtype)
    return (jnp.concatenate([src, loop]), jnp.concatenate([dst, loop]),
            jnp.concatenate([eattr, loop_attr], axis=0))


def _gat_conv(x, src, dst, eattr, W, a_s, a_d, We, a_e, b):
    n = x.shape[0]
    src, dst, eattr = _add_self_loops(src, dst, eattr, n)
    xw = (x @ W).reshape(n, HEAD, HID)
    al_s = jnp.sum(xw * a_s[None], axis=-1)
    al_d = jnp.sum(xw * a_d[None], axis=-1)
    al_e = jnp.sum((eattr @ We).reshape(-1, HEAD, HID) * a_e[None], axis=-1)
    al = al_s[src] + al_d[dst] + al_e
    al = jax.nn.leaky_relu(al, 0.2)
    amax = jax.ops.segment_max(al, dst, num_segments=n)
    ex = jnp.exp(al - amax[dst])
    denom = jax.ops.segment_sum(ex, dst, num_segments=n)
    p = ex / (denom[dst] + 1e-16)
    out = jax.ops.segment_sum(xw[src] * p[..., None], dst, num_segments=n)
    return out.reshape(n, F) + b


def kernel(x, edge_index, edge_weight, pre_W, pre_b, W1, att_src1, att_dst1,
           We1, att_edge1, b1, W2, att_src2, att_dst2, We2, att_edge2, b2,
           lin1_W, lin1_b, lin2_W, lin2_b, cls_W, cls_b):
    src, dst = edge_index[0], edge_index[1]
    h = jax.nn.relu(x @ pre_W + pre_b)
    h = h.reshape(-1, RD)
    h = jax.nn.leaky_relu(
        _gat_conv(h, src, dst, edge_weight, W1, att_src1, att_dst1, We1,
                  att_edge1, b1), 0.01)
    h = jax.nn.leaky_relu(
        _gat_conv(h, src, dst, edge_weight, W2, att_src2, att_dst2, We2,
                  att_edge2, b2), 0.01)
    h = h.reshape(B, NUM_NODE, F)
    h = h.reshape(B, NUM_NODE // POOL, POOL, F).max(axis=2)
    h = h.reshape(B, -1)
    return _dense_head(h, lin1_W, lin1_b, lin2_W, lin2_b, cls_W, cls_b)


# speedup vs baseline: 1.0000x; 1.0000x over previous
import functools

import jax
import jax.numpy as jnp
from jax.experimental import pallas as pl
from jax.experimental.pallas import tpu as pltpu

B, NUM_NODE, HID, HEAD, RD = 4, 12500, 6, 8, 8
N = B * NUM_NODE
POOL = 100
NUM_CLASSES = 33
F = HEAD * HID
LIN_IN = (NUM_NODE // POOL) * F
K_BLK = 1000
K_STEPS = LIN_IN // K_BLK
H1 = LIN_IN // 4


def _head_body(ht_blk, w1_blk, lin1_b, lin2_W, lin2_b, cls_W, cls_b, out_ref, acc):
    k = pl.program_id(0)

    @pl.when(k == 0)
    def _():
        acc[...] = jnp.zeros_like(acc)

    acc[...] += jax.lax.dot_general(
        w1_blk[...], ht_blk[...], (((0,), (0,)), ((), ())),
        preferred_element_type=jnp.float32)

    @pl.when(k == K_STEPS - 1)
    def _():
        h1 = jnp.maximum(acc[...] + lin1_b[...], 0.0)
        h2 = jnp.maximum(
            jax.lax.dot_general(h1, lin2_W[...], (((0,), (0,)), ((), ())),
                                preferred_element_type=jnp.float32)
            + lin2_b[...], 0.0)
        logits = jnp.dot(h2, cls_W[...],
                         preferred_element_type=jnp.float32) + cls_b[...]
        m = jnp.max(logits, axis=1, keepdims=True)
        z = logits - m
        lse = jnp.log(jnp.sum(jnp.exp(z), axis=1, keepdims=True))
        out_ref[...] = z - lse


def _dense_head(h, lin1_W, lin1_b, lin2_W, lin2_b, cls_W, cls_b):
    ht = h.T
    lin1_b = lin1_b.reshape(-1, 1)
    lin2_b = lin2_b.reshape(1, -1)
    cls_b = cls_b.reshape(1, -1)
    return pl.pallas_call(
        _head_body,
        grid=(K_STEPS,),
        in_specs=[
            pl.BlockSpec((K_BLK, B), lambda k: (k, 0)),
            pl.BlockSpec((K_BLK, H1), lambda k: (k, 0)),
            pl.BlockSpec((H1, 1), lambda k: (0, 0)),
            pl.BlockSpec((H1, 64), lambda k: (0, 0)),
            pl.BlockSpec((1, 64), lambda k: (0, 0)),
            pl.BlockSpec((64, NUM_CLASSES), lambda k: (0, 0)),
            pl.BlockSpec((1, NUM_CLASSES), lambda k: (0, 0)),
        ],
        out_specs=pl.BlockSpec((B, NUM_CLASSES), lambda k: (0, 0)),
        out_shape=jax.ShapeDtypeStruct((B, NUM_CLASSES), jnp.float32),
        scratch_shapes=[pltpu.VMEM((H1, B), jnp.float32)],
    )(ht, lin1_W, lin1_b, lin2_W, lin2_b, cls_W, cls_b)


def _add_self_loops(src, dst, eattr, n):
    ones = jnp.ones((src.shape[0],), dtype=eattr.dtype)
    cnt = jax.ops.segment_sum(ones, dst, num_segments=n)
    sums = jax.ops.segment_sum(eattr, dst, num_segments=n)
    loop_attr = sums / jnp.maximum(cnt, 1.0)[:, None]
    loop = jnp.arange(n, dtype=src.dtype)
    return (jnp.concatenate([src, loop]), jnp.concatenate([dst, loop]),
            jnp.concatenate([eattr, loop_attr], axis=0))


def _gat_conv(x, src, dst, eattr, W, a_s, a_d, We, a_e, b):
    n = x.shape[0]
    src, dst, eattr = _add_self_loops(src, dst, eattr, n)
    xw = (x @ W).reshape(n, HEAD, HID)
    al_s = jnp.sum(xw * a_s[None], axis=-1)
    al_d = jnp.sum(xw * a_d[None], axis=-1)
    al_e = jnp.sum((eattr @ We).reshape(-1, HEAD, HID) * a_e[None], axis=-1)
    al = al_s[src] + al_d[dst] + al_e
    al = jax.nn.leaky_relu(al, 0.2)
    amax = jax.ops.segment_max(al, dst, num_segments=n)
    ex = jnp.exp(al - amax[dst])
    denom = jax.ops.segment_sum(ex, dst, num_segments=n)
    p = ex / (denom[dst] + 1e-16)
    out = jax.ops.segment_sum(xw[src] * p[..., None], dst, num_segments=n)
    return out.reshape(n, F) + b


def kernel(x, edge_index, edge_weight, pre_W, pre_b, W1, att_src1, att_dst1,
           We1, att_edge1, b1, W2, att_src2, att_dst2, We2, att_edge2, b2,
           lin1_W, lin1_b, lin2_W, lin2_b, cls_W, cls_b):
    src, dst = edge_index[0], edge_index[1]
    h = jax.nn.relu(x @ pre_W + pre_b)
    h = h.reshape(-1, RD)
    h = jax.nn.leaky_relu(
        _gat_conv(h, src, dst, edge_weight, W1, att_src1, att_dst1, We1,
                  att_edge1, b1), 0.01)
    h = jax.nn.leaky_relu(
        _gat_conv(h, src, dst, edge_weight, W2, att_src2, att_dst2, We2,
                  att_edge2, b2), 0.01)
    h = h.reshape(B, NUM_NODE, F)
    h = h.reshape(B, NUM_NODE // POOL, POOL, F).max(axis=2)
    h = h.reshape(B, -1)
    return _dense_head(h, lin1_W, lin1_b, lin2_W, lin2_b, cls_W, cls_b)


# speedup vs baseline: 38.6871x; 38.6863x over previous
import jax
import jax.numpy as jnp
from jax import lax
from jax.experimental import pallas as pl
from jax.experimental.pallas import tpu as pltpu
from jax.experimental.pallas import tpu_sc as plsc

B, NUM_NODE, HID, HEAD, RD = 4, 12500, 6, 8, 8
N = B * NUM_NODE
E = 800000
POOL = 100
NUM_CLASSES = 33
F = HEAD * HID
LIN_IN = (NUM_NODE // POOL) * F
H1 = LIN_IN // 4

L = 16
NT = 16
HHEAD = HEAD // 2
NQ = 4

ECH = 512
NE_CH = -(-E // ECH)
E_PAD = NE_CH * ECH
NCH = 512
N_PAD = 50176
NN_CH = N_PAD // NCH
G_E = ECH // L
G_N = NCH // L


def _gat_sc_body(ei, ew, asd_lo, asd_hi, xws, ceU,
                 outq, exloop, exbuf,
                 idxa, idxb, ewv, ga, gb, xwg, exv, cwv, dgv, contrib,
                 ceUv, nasd, ncw, ndv, nexl, nout, nxw,
                 tab_sp, denom_sp, cw_sp, outacc_sp, sem):
    c = lax.axis_index("c")
    s = lax.axis_index("s")
    iota = lax.broadcasted_iota(jnp.int32, (L,), 0)
    nb_e = (NE_CH - 1 - s) // NT + 1
    nb_n = (NN_CH - 1 - s) // NT + 1

    pltpu.sync_copy(ceU, ceUv)

    def fulli(v):
        return jnp.full((L,), v, dtype=jnp.int32)

    def fullf(v):
        return jnp.full((L,), v, dtype=jnp.float32)

    def zero2d(ref, cols):
        def body(g, _):
            rows = (g // cols) * L + iota
            plsc.store_scatter(ref, [rows, fulli(g % cols)], fullf(0.0))
            return 0
        lax.fori_loop(0, (NCH // L) * cols, body, 0)

    @pl.when(s == 0)
    def _():
        @pl.when(c == 0)
        def _():
            pltpu.sync_copy(asd_lo, tab_sp)

        @pl.when(c == 1)
        def _():
            pltpu.sync_copy(asd_hi, tab_sp)

    zero2d(ndv, HHEAD)
    zero2d(ncw, 2)

    def ones_col0(g, _):
        plsc.store_scatter(cwv, [g * L + iota, fulli(0)], fullf(1.0))
        return 0
    lax.fori_loop(0, G_E, ones_col0, 0)

    def p0(k, _):
        nds = pl.ds((s + k * NT) * NCH, NCH)
        pltpu.sync_copy(ndv, denom_sp.at[nds])
        pltpu.sync_copy(ncw, cw_sp.at[nds])
        return 0
    lax.fori_loop(0, nb_n, p0, 0)
    plsc.subcore_barrier()

    def p1(k, _):
        base = (s + k * NT) * ECH
        eds = pl.ds(base, ECH)
        pltpu.sync_copy(ei.at[0, eds], idxa)
        pltpu.sync_copy(ei.at[1, eds], idxb)
        pltpu.sync_copy(ew.at[eds], ewv)
        pltpu.async_copy(tab_sp.at[idxa], ga, sem).wait()
        pltpu.async_copy(tab_sp.at[idxb], gb, sem).wait()

        def grp(g, _):
            rows = g * L + iota
            e = ewv[pl.ds(g * L, L)]
            for h in range(HHEAD):
                hv = fulli(h)
                gv = fulli(HHEAD) * c + fulli(h)
                sv = plsc.load_gather(ga, [rows, hv])
                dv = plsc.load_gather(gb, [rows, hv + HHEAD])
                ceh = plsc.load_gather(ceUv, [gv])
                uh = plsc.load_gather(ceUv, [gv + HEAD])
                al = sv + dv + e * ceh
                al = jnp.where(al >= 0.0, al, al * 0.2)
                ex = jnp.exp(al - uh)
                plsc.store_scatter(exv, [rows, hv], ex)
            plsc.store_scatter(cwv, [rows, fulli(1)], e)
            return 0
        lax.fori_loop(0, G_E, grp, 0)

        pltpu.sync_copy(exv, exbuf.at[c, eds])
        pltpu.sync_copy(exv, denom_sp.at[idxb], add=True)
        pltpu.sync_copy(cwv, cw_sp.at[idxb], add=True)
        return 0
    lax.fori_loop(0, nb_e, p1, 0)
    plsc.subcore_barrier()

    def p2(k, _):
        nds = pl.ds((s + k * NT) * NCH, NCH)
        pltpu.sync_copy(tab_sp.at[nds], nasd)
        pltpu.sync_copy(cw_sp.at[nds], ncw)
        pltpu.sync_copy(denom_sp.at[nds], ndv)

        def grp(g, _):
            rows = g * L + iota
            cnt = plsc.load_gather(ncw, [rows, fulli(0)])
            ews = plsc.load_gather(ncw, [rows, fulli(1)])
            la = ews / jnp.maximum(cnt, 1.0)
            for h in range(HHEAD):
                hv = fulli(h)
                gv = fulli(HHEAD) * c + fulli(h)
                sv = plsc.load_gather(nasd, [rows, hv])
                dv = plsc.load_gather(nasd, [rows, hv + HHEAD])
                ceh = plsc.load_gather(ceUv, [gv])
                uh = plsc.load_gather(ceUv, [gv + HEAD])
                al = sv + dv + la * ceh
                al = jnp.where(al >= 0.0, al, al * 0.2)
                ex = jnp.exp(al - uh)
                plsc.store_scatter(nexl, [rows, hv], ex)
                old = plsc.load_gather(ndv, [rows, hv])
                plsc.store_scatter(ndv, [rows, hv], old + ex)
            return 0
        lax.fori_loop(0, G_N, grp, 0)

        pltpu.sync_copy(ndv, denom_sp.at[nds])
        pltpu.sync_copy(nexl, exloop.at[c, nds])
        return 0
    lax.fori_loop(0, nb_n, p2, 0)
    plsc.subcore_barrier()

    def subpass(q, _):
        @pl.when(s == 0)
        def _():
            pltpu.sync_copy(xws.at[c, q], tab_sp)

        zero2d(nout, HID)

        def z(k, _):
            pltpu.sync_copy(nout, outacc_sp.at[pl.ds((s + k * NT) * NCH, NCH)])
            return 0
        lax.fori_loop(0, nb_n, z, 0)
        plsc.subcore_barrier()

        def p3(k, _):
            base = (s + k * NT) * ECH
            eds = pl.ds(base, ECH)
            pltpu.sync_copy(ei.at[0, eds], idxa)
            pltpu.sync_copy(ei.at[1, eds], idxb)
            pltpu.sync_copy(exbuf.at[c, eds], exv)
            pltpu.async_copy(tab_sp.at[idxa], xwg, sem).wait()
            pltpu.async_copy(denom_sp.at[idxb], dgv, sem).wait()

            def grp(g, _):
                rows = g * L + iota
                qv = fulli(q)
                exh = plsc.load_gather(exv, [rows, qv])
                dg = plsc.load_gather(dgv, [rows, qv])
                ps = exh / dg
                for j in range(HID):
                    jv = fulli(j)
                    col = plsc.load_gather(xwg, [rows, jv])
                    plsc.store_scatter(contrib, [rows, jv], col * ps)
                return 0
            lax.fori_loop(0, G_E, grp, 0)

            pltpu.sync_copy(contrib, outacc_sp.at[idxb], add=True)
            return 0
        lax.fori_loop(0, nb_e, p3, 0)
        plsc.subcore_barrier()

        def p4(k, _):
            nds = pl.ds((s + k * NT) * NCH, NCH)
            pltpu.sync_copy(outacc_sp.at[nds], nout)
            pltpu.sync_copy(denom_sp.at[nds], ndv)
            pltpu.sync_copy(exloop.at[c, nds], nexl)
            pltpu.sync_copy(tab_sp.at[nds], nxw)

            def grp(g, _):
                rows = g * L + iota
                qv = fulli(q)
                exh = plsc.load_gather(nexl, [rows, qv])
                dg = plsc.load_gather(ndv, [rows, qv])
                ps = exh / dg
                for j in range(HID):
                    jv = fulli(j)
                    col = plsc.load_gather(nxw, [rows, jv])
                    o = plsc.load_gather(nout, [rows, jv])
                    plsc.store_scatter(nout, [rows, jv], o + col * ps)
                return 0
            lax.fori_loop(0, G_N, grp, 0)

            pltpu.sync_copy(nout, outq.at[c, q, nds])
            return 0
        lax.fori_loop(0, nb_n, p4, 0)
        plsc.subcore_barrier()
        return 0

    lax.fori_loop(0, NQ, subpass, 0)


def _gat_sc(ei_p, ew_p, asd_lo, asd_hi, xws, ceU):
    mesh = plsc.VectorSubcoreMesh(core_axis_name="c", subcore_axis_name="s")
    f = pl.kernel(
        _gat_sc_body,
        out_type=(
            jax.ShapeDtypeStruct((2, NQ, N_PAD, HID), jnp.float32),
            jax.ShapeDtypeStruct((2, N_PAD, HHEAD), jnp.float32),
            jax.ShapeDtypeStruct((2, E_PAD, HHEAD), jnp.float32),
        ),
        mesh=mesh,
        scratch_types=[
            pltpu.VMEM((ECH,), jnp.int32),
            pltpu.VMEM((ECH,), jnp.int32),
            pltpu.VMEM((ECH,), jnp.float32),
            pltpu.VMEM((ECH, 2 * HHEAD), jnp.float32),
            pltpu.VMEM((ECH, 2 * HHEAD), jnp.float32),
            pltpu.VMEM((ECH, 2 * HHEAD), jnp.float32),
            pltpu.VMEM((ECH, HHEAD), jnp.float32),
            pltpu.VMEM((ECH, 2), jnp.float32),
            pltpu.VMEM((ECH, HHEAD), jnp.float32),
            pltpu.VMEM((ECH, HID), jnp.float32),
            pltpu.VMEM((L,), jnp.float32),
            pltpu.VMEM((NCH, 2 * HHEAD), jnp.float32),
            pltpu.VMEM((NCH, 2), jnp.float32),
            pltpu.VMEM((NCH, HHEAD), jnp.float32),
            pltpu.VMEM((NCH, HHEAD), jnp.float32),
            pltpu.VMEM((NCH, HID), jnp.float32),
            pltpu.VMEM((NCH, 2 * HHEAD), jnp.float32),
            pltpu.VMEM_SHARED((N_PAD, 2 * HHEAD), jnp.float32),
            pltpu.VMEM_SHARED((N_PAD, HHEAD), jnp.float32),
            pltpu.VMEM_SHARED((N_PAD, 2), jnp.float32),
            pltpu.VMEM_SHARED((N_PAD, HID), jnp.float32),
            pltpu.SemaphoreType.DMA,
        ],
        compiler_params=pltpu.CompilerParams(
            use_tc_tiling_on_sc=False, needs_layout_passes=False),
    )
    outq = f(ei_p, ew_p, asd_lo, asd_hi, xws, ceU)[0]
    return list(outq.reshape(HEAD, N_PAD, HID))


RB = 1792


def _sel_mat():
    r = lax.broadcasted_iota(jnp.int32, (F, HEAD), 0)
    cj = lax.broadcasted_iota(jnp.int32, (F, HEAD), 1)
    return (r // HID == cj).astype(jnp.float32)


def _emit_outs(xw, a_s, a_d, outs, mx, k):
    z = jnp.zeros((RB, 2), jnp.float32)
    for h in range(HEAD):
        outs[h][...] = jnp.concatenate(
            [xw[:, h * HID:(h + 1) * HID], z], axis=1)
    outs[HEAD][...] = jnp.concatenate(
        [a_s[:, :HHEAD], a_d[:, :HHEAD]], axis=1)
    outs[HEAD + 1][...] = jnp.concatenate(
        [a_s[:, HHEAD:], a_d[:, HHEAD:]], axis=1)
    bm = jnp.max(jnp.concatenate([a_s, a_d], axis=1), axis=0, keepdims=True)

    @pl.when(k == 0)
    def _():
        mx[...] = bm

    @pl.when(k != 0)
    def _():
        mx[...] = jnp.maximum(mx[...], bm)


def _p1_body(x_blk, preW, preb, W1, asf, adf, *outs):
    k = pl.program_id(0)
    h0 = jnp.maximum(x_blk[...] * preW[...] + preb[...], 0.0)
    xw = jnp.dot(h0, W1[...], preferred_element_type=jnp.float32)
    S = _sel_mat()
    a_s = jnp.dot(xw * asf[...], S, preferred_element_type=jnp.float32)
    a_d = jnp.dot(xw * adf[...], S, preferred_element_type=jnp.float32)
    _emit_outs(xw, a_s, a_d, outs[:-1], outs[-1], k)


_NODE_OUTS = (
    [jax.ShapeDtypeStruct((N_PAD, 2 * HHEAD), jnp.float32)] * (HEAD + 2)
    + [jax.ShapeDtypeStruct((1, 2 * HEAD), jnp.float32)]
)

_NODE_OUT_SPECS = (
    [pl.BlockSpec((RB, 2 * HHEAD), lambda k: (k, 0))] * (HEAD + 2)
    + [pl.BlockSpec((1, 2 * HEAD), lambda k: (0, 0))]
)


def _p1(x_p, pre_W, pre_b, W1, asf, adf):
    return pl.pallas_call(
        _p1_body,
        grid=(N_PAD // RB,),
        in_specs=[
            pl.BlockSpec((RB, 1), lambda k: (k, 0)),
            pl.BlockSpec((1, RD), lambda k: (0, 0)),
            pl.BlockSpec((1, RD), lambda k: (0, 0)),
            pl.BlockSpec((RD, F), lambda k: (0, 0)),
            pl.BlockSpec((1, F), lambda k: (0, 0)),
            pl.BlockSpec((1, F), lambda k: (0, 0)),
        ],
        out_specs=_NODE_OUT_SPECS,
        out_shape=_NODE_OUTS,
    )(x_p, pre_W, pre_b, W1, asf, adf)


def _mid_body(o0, o1, o2, o3, o4, o5, o6, o7, b1r, W2f, asf, adf, *outs):
    k = pl.program_id(0)
    o = jnp.concatenate([o0[...], o1[...], o2[...], o3[...],
                         o4[...], o5[...], o6[...], o7[...]], axis=1)
    h1 = o + b1r[...]
    h1 = jnp.where(h1 >= 0.0, h1, 0.01 * h1)
    xw = jnp.dot(h1, W2f[...], preferred_element_type=jnp.float32)
    S = _sel_mat()
    a_s = jnp.dot(xw * asf[...], S, preferred_element_type=jnp.float32)
    a_d = jnp.dot(xw * adf[...], S, preferred_element_type=jnp.float32)
    _emit_outs(xw, a_s, a_d, outs[:-1], outs[-1], k)


def _mid(outs, b1, W2, asf, adf):
    return pl.pallas_call(
        _mid_body,
        grid=(N_PAD // RB,),
        in_specs=(
            [pl.BlockSpec((RB, HID), lambda k: (k, 0))] * HEAD
            + [pl.BlockSpec((1, F), lambda k: (0, 0)),
               pl.BlockSpec((F, F), lambda k: (0, 0)),
               pl.BlockSpec((1, F), lambda k: (0, 0)),
               pl.BlockSpec((1, F), lambda k: (0, 0))]
        ),
        out_specs=_NODE_OUT_SPECS,
        out_shape=_NODE_OUTS,
    )(*outs, b1.reshape(1, F), W2, asf, adf)


def _comb_body(v0, v1, v2, v3, v4, v5, v6, v7, hout):
    hout[...] = jnp.concatenate([v0[...], v1[...], v2[...], v3[...],
                                 v4[...], v5[...], v6[...], v7[...]], axis=1)


def _combine(vq):
    return pl.pallas_call(
        _comb_body,
        grid=(N_PAD // RB,),
        in_specs=[pl.BlockSpec((RB, HID), lambda k: (k, 0))] * HEAD,
        out_specs=pl.BlockSpec((RB, F), lambda k: (k, 0)),
        out_shape=jax.ShapeDtypeStruct((N_PAD, F), jnp.float32),
    )(*vq)


PG = 100


def _pool_body(v, b2r, pout):
    k = pl.program_id(0)
    m = jnp.max(v[...], axis=1) + b2r[...]
    pout[pl.ds(k * PG, PG), :] = jnp.where(m >= 0.0, m, 0.01 * m)


def _pool(v3d, b2):
    return pl.pallas_call(
        _pool_body,
        grid=(500 // PG,),
        in_specs=[pl.BlockSpec((PG, POOL, F), lambda k: (k, 0, 0)),
                  pl.BlockSpec((1, F), lambda k: (0, 0))],
        out_specs=pl.BlockSpec((500, F), lambda k: (0, 0)),
        out_shape=jax.ShapeDtypeStruct((500, F), jnp.float32),
    )(v3d, b2.reshape(1, F))


K_BLK = 1000
K_STEPS = LIN_IN // K_BLK


def _head_body(ht_blk, w1_blk, lin1_b, lin2_W, lin2_b, cls_W, cls_b, out_ref, acc):
    k = pl.program_id(0)

    @pl.when(k == 0)
    def _():
        acc[...] = jnp.zeros_like(acc)

    acc[...] += jax.lax.dot_general(
        w1_blk[...], ht_blk[...], (((0,), (0,)), ((), ())),
        preferred_element_type=jnp.float32)

    @pl.when(k == K_STEPS - 1)
    def _():
        h1 = jnp.maximum(acc[...] + lin1_b[...], 0.0)
        h2 = jnp.maximum(
            jax.lax.dot_general(h1, lin2_W[...], (((0,), (0,)), ((), ())),
                                preferred_element_type=jnp.float32)
            + lin2_b[...], 0.0)
        logits = jnp.dot(h2, cls_W[...],
                         preferred_element_type=jnp.float32) + cls_b[...]
        m = jnp.max(logits, axis=1, keepdims=True)
        z = logits - m
        lse = jnp.log(jnp.sum(jnp.exp(z), axis=1, keepdims=True))
        out_ref[...] = z - lse


def _dense_head(h, lin1_W, lin1_b, lin2_W, lin2_b, cls_W, cls_b):
    ht = h.T
    return pl.pallas_call(
        _head_body,
        grid=(K_STEPS,),
        in_specs=[
            pl.BlockSpec((K_BLK, B), lambda k: (k, 0)),
            pl.BlockSpec((K_BLK, H1), lambda k: (k, 0)),
            pl.BlockSpec((H1, 1), lambda k: (0, 0)),
            pl.BlockSpec((H1, 64), lambda k: (0, 0)),
            pl.BlockSpec((1, 64), lambda k: (0, 0)),
            pl.BlockSpec((64, NUM_CLASSES), lambda k: (0, 0)),
            pl.BlockSpec((1, NUM_CLASSES), lambda k: (0, 0)),
        ],
        out_specs=pl.BlockSpec((B, NUM_CLASSES), lambda k: (0, 0)),
        out_shape=jax.ShapeDtypeStruct((B, NUM_CLASSES), jnp.float32),
        scratch_shapes=[pltpu.VMEM((H1, B), jnp.float32)],
    )(ht, lin1_W, lin1_b.reshape(-1, 1), lin2_W, lin2_b.reshape(1, -1),
      cls_W, cls_b.reshape(1, -1))


def _ceU(We, a_e, mx):
    ce = (We.reshape(HEAD, HID) * a_e).sum(axis=1)
    u = mx[0, :HEAD] + mx[0, HEAD:] + jnp.maximum(ce, 0.0)
    u = jnp.where(u >= 0.0, u, 0.2 * u)
    return jnp.concatenate([ce, u]).astype(jnp.float32)


def kernel(x, edge_index, edge_weight, pre_W, pre_b, W1, att_src1, att_dst1,
           We1, att_edge1, b1, W2, att_src2, att_dst2, We2, att_edge2, b2,
           lin1_W, lin1_b, lin2_W, lin2_b, cls_W, cls_b):
    pad_e = E_PAD - E
    ei_p = jnp.concatenate(
        [edge_index, jnp.full((2, pad_e), N, dtype=edge_index.dtype)], axis=1)
    ew_p = jnp.concatenate(
        [edge_weight[:, 0], jnp.zeros((pad_e,), jnp.float32)])

    x_p = jnp.concatenate(
        [x.reshape(N, 1), jnp.zeros((N_PAD - N, 1), jnp.float32)], axis=0)

    *qs, alo, ahi, mx1 = _p1(
        x_p, pre_W.reshape(1, RD), pre_b.reshape(1, RD), W1,
        att_src1.reshape(1, F), att_dst1.reshape(1, F))
    ceU1 = _ceU(We1, att_edge1, mx1)
    outs1 = _gat_sc(ei_p, ew_p, alo, ahi,
                    jnp.stack(qs).reshape(2, NQ, N_PAD, 2 * HHEAD), ceU1)

    *qs, alo, ahi, mx2 = _mid(
        outs1, b1, W2, att_src2.reshape(1, F), att_dst2.reshape(1, F))
    ceU2 = _ceU(We2, att_edge2, mx2)
    outs2 = _gat_sc(ei_p, ew_p, alo, ahi,
                    jnp.stack(qs).reshape(2, NQ, N_PAD, 2 * HHEAD), ceU2)

    h2 = _combine(outs2)[:N].reshape(500, POOL, F)
    h = _pool(h2, b2).reshape(B, LIN_IN)
    return _dense_head(h, lin1_W, lin1_b, lin2_W, lin2_b, cls_W, cls_b)


# speedup vs baseline: 46.7621x; 1.2087x over previous
import jax
import jax.numpy as jnp
from jax import lax
from jax.experimental import pallas as pl
from jax.experimental.pallas import tpu as pltpu
from jax.experimental.pallas import tpu_sc as plsc

B, NUM_NODE, HID, HEAD, RD = 4, 12500, 6, 8, 8
N = B * NUM_NODE
E = 800000
POOL = 100
NUM_CLASSES = 33
F = HEAD * HID
LIN_IN = (NUM_NODE // POOL) * F
H1 = LIN_IN // 4

L = 16
NT = 16
HHEAD = HEAD // 2
NQ = 4

ECH = 512
NE_CH = -(-E // ECH)
E_PAD = NE_CH * ECH
NCH = 512
N_PAD = 50176
NN_CH = N_PAD // NCH
G_E = ECH // L
G_N = NCH // L


def _gat_sc_body(ei, ew, asd_lo, asd_hi, xws, ceU,
                 outq, exloop, exbuf,
                 idxa, idxb, ewv, ga, gb, xwg, exv, cwv, dgv, contrib,
                 ceUv, nasd, ncw, ndv, nexl, nout, nxw,
                 tab_sp, denom_sp, cw_sp, outacc_sp, sem):
    c = lax.axis_index("c")
    s = lax.axis_index("s")
    iota = lax.broadcasted_iota(jnp.int32, (L,), 0)
    nb_e = (NE_CH - 1 - s) // NT + 1
    nb_n = (NN_CH - 1 - s) // NT + 1

    pltpu.sync_copy(ceU, ceUv)

    def fulli(v):
        return jnp.full((L,), v, dtype=jnp.int32)

    def fullf(v):
        return jnp.full((L,), v, dtype=jnp.float32)

    def zero2d(ref, cols):
        def body(g, _):
            rows = (g // cols) * L + iota
            plsc.store_scatter(ref, [rows, fulli(g % cols)], fullf(0.0))
            return 0
        lax.fori_loop(0, (NCH // L) * cols, body, 0)

    @pl.when(s == 0)
    def _():
        @pl.when(c == 0)
        def _():
            pltpu.sync_copy(asd_lo, tab_sp)

        @pl.when(c == 1)
        def _():
            pltpu.sync_copy(asd_hi, tab_sp)

    zero2d(ndv, HHEAD)
    zero2d(ncw, 2)

    def ones_col0(g, _):
        plsc.store_scatter(cwv, [g * L + iota, fulli(0)], fullf(1.0))
        return 0
    lax.fori_loop(0, G_E, ones_col0, 0)

    def p0(k, _):
        nds = pl.ds((s + k * NT) * NCH, NCH)
        pltpu.sync_copy(ndv, denom_sp.at[nds])
        pltpu.sync_copy(ncw, cw_sp.at[nds])
        return 0
    lax.fori_loop(0, nb_n, p0, 0)
    plsc.subcore_barrier()

    def p1(k, _):
        base = (s + k * NT) * ECH
        eds = pl.ds(base, ECH)
        d1 = pltpu.async_copy(ei.at[0, eds], idxa, sem)
        d2 = pltpu.async_copy(ei.at[1, eds], idxb, sem)
        d3 = pltpu.async_copy(ew.at[eds], ewv, sem)
        d1.wait()
        d2.wait()
        d3.wait()
        g1 = pltpu.async_copy(tab_sp.at[idxa], ga, sem)
        g2 = pltpu.async_copy(tab_sp.at[idxb], gb, sem)
        g1.wait()
        g2.wait()

        def grp(g, _):
            rows = g * L + iota
            e = ewv[pl.ds(g * L, L)]
            for h in range(HHEAD):
                hv = fulli(h)
                gv = fulli(HHEAD) * c + fulli(h)
                sv = plsc.load_gather(ga, [rows, hv])
                dv = plsc.load_gather(gb, [rows, hv + HHEAD])
                ceh = plsc.load_gather(ceUv, [gv])
                uh = plsc.load_gather(ceUv, [gv + HEAD])
                al = sv + dv + e * ceh
                al = jnp.where(al >= 0.0, al, al * 0.2)
                ex = jnp.exp(al - uh)
                plsc.store_scatter(exv, [rows, hv], ex)
            plsc.store_scatter(cwv, [rows, fulli(1)], e)
            return 0
        lax.fori_loop(0, G_E, grp, 0)

        pltpu.sync_copy(exv, exbuf.at[c, eds])
        pltpu.sync_copy(exv, denom_sp.at[idxb], add=True)
        pltpu.sync_copy(cwv, cw_sp.at[idxb], add=True)
        return 0
    lax.fori_loop(0, nb_e, p1, 0)
    plsc.subcore_barrier()

    def p2(k, _):
        nds = pl.ds((s + k * NT) * NCH, NCH)
        pltpu.sync_copy(tab_sp.at[nds], nasd)
        pltpu.sync_copy(cw_sp.at[nds], ncw)
        pltpu.sync_copy(denom_sp.at[nds], ndv)

        def grp(g, _):
            rows = g * L + iota
            cnt = plsc.load_gather(ncw, [rows, fulli(0)])
            ews = plsc.load_gather(ncw, [rows, fulli(1)])
            la = ews / jnp.maximum(cnt, 1.0)
            for h in range(HHEAD):
                hv = fulli(h)
                gv = fulli(HHEAD) * c + fulli(h)
                sv = plsc.load_gather(nasd, [rows, hv])
                dv = plsc.load_gather(nasd, [rows, hv + HHEAD])
                ceh = plsc.load_gather(ceUv, [gv])
                uh = plsc.load_gather(ceUv, [gv + HEAD])
                al = sv + dv + la * ceh
                al = jnp.where(al >= 0.0, al, al * 0.2)
                ex = jnp.exp(al - uh)
                plsc.store_scatter(nexl, [rows, hv], ex)
                old = plsc.load_gather(ndv, [rows, hv])
                plsc.store_scatter(ndv, [rows, hv], old + ex)
            return 0
        lax.fori_loop(0, G_N, grp, 0)

        pltpu.sync_copy(ndv, denom_sp.at[nds])
        pltpu.sync_copy(nexl, exloop.at[c, nds])
        return 0
    lax.fori_loop(0, nb_n, p2, 0)
    plsc.subcore_barrier()

    def subpass(q, _):
        @pl.when(s == 0)
        def _():
            pltpu.sync_copy(xws.at[c, q], tab_sp)

        zero2d(nout, HID)

        def z(k, _):
            pltpu.sync_copy(nout, outacc_sp.at[pl.ds((s + k * NT) * NCH, NCH)])
            return 0
        lax.fori_loop(0, nb_n, z, 0)
        plsc.subcore_barrier()

        def p3(k, _):
            base = (s + k * NT) * ECH
            eds = pl.ds(base, ECH)
            d1 = pltpu.async_copy(ei.at[0, eds], idxa, sem)
            d2 = pltpu.async_copy(ei.at[1, eds], idxb, sem)
            d3 = pltpu.async_copy(exbuf.at[c, eds], exv, sem)
            d1.wait()
            d2.wait()
            d3.wait()
            g1 = pltpu.async_copy(tab_sp.at[idxa], xwg, sem)
            g2 = pltpu.async_copy(denom_sp.at[idxb], dgv, sem)
            g1.wait()
            g2.wait()

            def grp(g, _):
                rows = g * L + iota
                qv = fulli(q)
                exh = plsc.load_gather(exv, [rows, qv])
                dg = plsc.load_gather(dgv, [rows, qv])
                ps = exh / dg
                for j in range(HID):
                    jv = fulli(j)
                    col = plsc.load_gather(xwg, [rows, jv])
                    plsc.store_scatter(contrib, [rows, jv], col * ps)
                return 0
            lax.fori_loop(0, G_E, grp, 0)

            pltpu.sync_copy(contrib, outacc_sp.at[idxb], add=True)
            return 0
        lax.fori_loop(0, nb_e, p3, 0)
        plsc.subcore_barrier()

        def p4(k, _):
            nds = pl.ds((s + k * NT) * NCH, NCH)
            pltpu.sync_copy(outacc_sp.at[nds], nout)
            pltpu.sync_copy(denom_sp.at[nds], ndv)
            pltpu.sync_copy(exloop.at[c, nds], nexl)
            pltpu.sync_copy(tab_sp.at[nds], nxw)

            def grp(g, _):
                rows = g * L + iota
                qv = fulli(q)
                exh = plsc.load_gather(nexl, [rows, qv])
                dg = plsc.load_gather(ndv, [rows, qv])
                ps = exh / dg
                for j in range(HID):
                    jv = fulli(j)
                    col = plsc.load_gather(nxw, [rows, jv])
                    o = plsc.load_gather(nout, [rows, jv])
                    plsc.store_scatter(nout, [rows, jv], o + col * ps)
                return 0
            lax.fori_loop(0, G_N, grp, 0)

            pltpu.sync_copy(nout, outq.at[c, q, nds])
            return 0
        lax.fori_loop(0, nb_n, p4, 0)
        plsc.subcore_barrier()
        return 0

    lax.fori_loop(0, NQ, subpass, 0)


def _gat_sc(ei_p, ew_p, asd_lo, asd_hi, xws, ceU):
    mesh = plsc.VectorSubcoreMesh(core_axis_name="c", subcore_axis_name="s")
    f = pl.kernel(
        _gat_sc_body,
        out_type=(
            jax.ShapeDtypeStruct((2, NQ, N_PAD, HID), jnp.float32),
            jax.ShapeDtypeStruct((2, N_PAD, HHEAD), jnp.float32),
            jax.ShapeDtypeStruct((2, E_PAD, HHEAD), jnp.float32),
        ),
        mesh=mesh,
        scratch_types=[
            pltpu.VMEM((ECH,), jnp.int32),
            pltpu.VMEM((ECH,), jnp.int32),
            pltpu.VMEM((ECH,), jnp.float32),
            pltpu.VMEM((ECH, 2 * HHEAD), jnp.float32),
            pltpu.VMEM((ECH, 2 * HHEAD), jnp.float32),
            pltpu.VMEM((ECH, 2 * HHEAD), jnp.float32),
            pltpu.VMEM((ECH, HHEAD), jnp.float32),
            pltpu.VMEM((ECH, 2), jnp.float32),
            pltpu.VMEM((ECH, HHEAD), jnp.float32),
            pltpu.VMEM((ECH, HID), jnp.float32),
            pltpu.VMEM((L,), jnp.float32),
            pltpu.VMEM((NCH, 2 * HHEAD), jnp.float32),
            pltpu.VMEM((NCH, 2), jnp.float32),
            pltpu.VMEM((NCH, HHEAD), jnp.float32),
            pltpu.VMEM((NCH, HHEAD), jnp.float32),
            pltpu.VMEM((NCH, HID), jnp.float32),
            pltpu.VMEM((NCH, 2 * HHEAD), jnp.float32),
            pltpu.VMEM_SHARED((N_PAD, 2 * HHEAD), jnp.float32),
            pltpu.VMEM_SHARED((N_PAD, HHEAD), jnp.float32),
            pltpu.VMEM_SHARED((N_PAD, 2), jnp.float32),
            pltpu.VMEM_SHARED((N_PAD, HID), jnp.float32),
            pltpu.SemaphoreType.DMA,
        ],
        compiler_params=pltpu.CompilerParams(
            use_tc_tiling_on_sc=False, needs_layout_passes=False),
    )
    outq = f(ei_p, ew_p, asd_lo, asd_hi, xws, ceU)[0]
    return list(outq.reshape(HEAD, N_PAD, HID))


RB = 1792


def _sel_mat():
    r = lax.broadcasted_iota(jnp.int32, (F, HEAD), 0)
    cj = lax.broadcasted_iota(jnp.int32, (F, HEAD), 1)
    return (r // HID == cj).astype(jnp.float32)


def _emit_outs(xw, a_s, a_d, outs, mx, k):
    z = jnp.zeros((RB, 2), jnp.float32)
    for h in range(HEAD):
        outs[h][...] = jnp.concatenate(
            [xw[:, h * HID:(h + 1) * HID], z], axis=1)
    outs[HEAD][...] = jnp.concatenate(
        [a_s[:, :HHEAD], a_d[:, :HHEAD]], axis=1)
    outs[HEAD + 1][...] = jnp.concatenate(
        [a_s[:, HHEAD:], a_d[:, HHEAD:]], axis=1)
    bm = jnp.max(jnp.concatenate([a_s, a_d], axis=1), axis=0, keepdims=True)

    @pl.when(k == 0)
    def _():
        mx[...] = bm

    @pl.when(k != 0)
    def _():
        mx[...] = jnp.maximum(mx[...], bm)


def _p1_body(x_blk, preW, preb, W1, asf, adf, *outs):
    k = pl.program_id(0)
    h0 = jnp.maximum(x_blk[...] * preW[...] + preb[...], 0.0)
    xw = jnp.dot(h0, W1[...], preferred_element_type=jnp.float32)
    S = _sel_mat()
    a_s = jnp.dot(xw * asf[...], S, preferred_element_type=jnp.float32)
    a_d = jnp.dot(xw * adf[...], S, preferred_element_type=jnp.float32)
    _emit_outs(xw, a_s, a_d, outs[:-1], outs[-1], k)


_NODE_OUTS = (
    [jax.ShapeDtypeStruct((N_PAD, 2 * HHEAD), jnp.float32)] * (HEAD + 2)
    + [jax.ShapeDtypeStruct((1, 2 * HEAD), jnp.float32)]
)

_NODE_OUT_SPECS = (
    [pl.BlockSpec((RB, 2 * HHEAD), lambda k: (k, 0))] * (HEAD + 2)
    + [pl.BlockSpec((1, 2 * HEAD), lambda k: (0, 0))]
)


def _p1(x_p, pre_W, pre_b, W1, asf, adf):
    return pl.pallas_call(
        _p1_body,
        grid=(N_PAD // RB,),
        in_specs=[
            pl.BlockSpec((RB, 1), lambda k: (k, 0)),
            pl.BlockSpec((1, RD), lambda k: (0, 0)),
            pl.BlockSpec((1, RD), lambda k: (0, 0)),
            pl.BlockSpec((RD, F), lambda k: (0, 0)),
            pl.BlockSpec((1, F), lambda k: (0, 0)),
            pl.BlockSpec((1, F), lambda k: (0, 0)),
        ],
        out_specs=_NODE_OUT_SPECS,
        out_shape=_NODE_OUTS,
    )(x_p, pre_W, pre_b, W1, asf, adf)


def _mid_body(o0, o1, o2, o3, o4, o5, o6, o7, b1r, W2f, asf, adf, *outs):
    k = pl.program_id(0)
    o = jnp.concatenate([o0[...], o1[...], o2[...], o3[...],
                         o4[...], o5[...], o6[...], o7[...]], axis=1)
    h1 = o + b1r[...]
    h1 = jnp.where(h1 >= 0.0, h1, 0.01 * h1)
    xw = jnp.dot(h1, W2f[...], preferred_element_type=jnp.float32)
    S = _sel_mat()
    a_s = jnp.dot(xw * asf[...], S, preferred_element_type=jnp.float32)
    a_d = jnp.dot(xw * adf[...], S, preferred_element_type=jnp.float32)
    _emit_outs(xw, a_s, a_d, outs[:-1], outs[-1], k)


def _mid(outs, b1, W2, asf, adf):
    return pl.pallas_call(
        _mid_body,
        grid=(N_PAD // RB,),
        in_specs=(
            [pl.BlockSpec((RB, HID), lambda k: (k, 0))] * HEAD
            + [pl.BlockSpec((1, F), lambda k: (0, 0)),
               pl.BlockSpec((F, F), lambda k: (0, 0)),
               pl.BlockSpec((1, F), lambda k: (0, 0)),
               pl.BlockSpec((1, F), lambda k: (0, 0))]
        ),
        out_specs=_NODE_OUT_SPECS,
        out_shape=_NODE_OUTS,
    )(*outs, b1.reshape(1, F), W2, asf, adf)


def _comb_body(v0, v1, v2, v3, v4, v5, v6, v7, hout):
    hout[...] = jnp.concatenate([v0[...], v1[...], v2[...], v3[...],
                                 v4[...], v5[...], v6[...], v7[...]], axis=1)


def _combine(vq):
    return pl.pallas_call(
        _comb_body,
        grid=(N_PAD // RB,),
        in_specs=[pl.BlockSpec((RB, HID), lambda k: (k, 0))] * HEAD,
        out_specs=pl.BlockSpec((RB, F), lambda k: (k, 0)),
        out_shape=jax.ShapeDtypeStruct((N_PAD, F), jnp.float32),
    )(*vq)


PG = 100


def _pool_body(v, b2r, pout):
    k = pl.program_id(0)
    m = jnp.max(v[...], axis=1) + b2r[...]
    pout[pl.ds(k * PG, PG), :] = jnp.where(m >= 0.0, m, 0.01 * m)


def _pool(v3d, b2):
    return pl.pallas_call(
        _pool_body,
        grid=(500 // PG,),
        in_specs=[pl.BlockSpec((PG, POOL, F), lambda k: (k, 0, 0)),
                  pl.BlockSpec((1, F), lambda k: (0, 0))],
        out_specs=pl.BlockSpec((500, F), lambda k: (0, 0)),
        out_shape=jax.ShapeDtypeStruct((500, F), jnp.float32),
    )(v3d, b2.reshape(1, F))


K_BLK = 1000
K_STEPS = LIN_IN // K_BLK


def _head_body(ht_blk, w1_blk, lin1_b, lin2_W, lin2_b, cls_W, cls_b, out_ref, acc):
    k = pl.program_id(0)

    @pl.when(k == 0)
    def _():
        acc[...] = jnp.zeros_like(acc)

    acc[...] += jax.lax.dot_general(
        w1_blk[...], ht_blk[...], (((0,), (0,)), ((), ())),
        preferred_element_type=jnp.float32)

    @pl.when(k == K_STEPS - 1)
    def _():
        h1 = jnp.maximum(acc[...] + lin1_b[...], 0.0)
        h2 = jnp.maximum(
            jax.lax.dot_general(h1, lin2_W[...], (((0,), (0,)), ((), ())),
                                preferred_element_type=jnp.float32)
            + lin2_b[...], 0.0)
        logits = jnp.dot(h2, cls_W[...],
                         preferred_element_type=jnp.float32) + cls_b[...]
        m = jnp.max(logits, axis=1, keepdims=True)
        z = logits - m
        lse = jnp.log(jnp.sum(jnp.exp(z), axis=1, keepdims=True))
        out_ref[...] = z - lse


def _dense_head(h, lin1_W, lin1_b, lin2_W, lin2_b, cls_W, cls_b):
    ht = h.T
    return pl.pallas_call(
        _head_body,
        grid=(K_STEPS,),
        in_specs=[
            pl.BlockSpec((K_BLK, B), lambda k: (k, 0)),
            pl.BlockSpec((K_BLK, H1), lambda k: (k, 0)),
            pl.BlockSpec((H1, 1), lambda k: (0, 0)),
            pl.BlockSpec((H1, 64), lambda k: (0, 0)),
            pl.BlockSpec((1, 64), lambda k: (0, 0)),
            pl.BlockSpec((64, NUM_CLASSES), lambda k: (0, 0)),
            pl.BlockSpec((1, NUM_CLASSES), lambda k: (0, 0)),
        ],
        out_specs=pl.BlockSpec((B, NUM_CLASSES), lambda k: (0, 0)),
        out_shape=jax.ShapeDtypeStruct((B, NUM_CLASSES), jnp.float32),
        scratch_shapes=[pltpu.VMEM((H1, B), jnp.float32)],
    )(ht, lin1_W, lin1_b.reshape(-1, 1), lin2_W, lin2_b.reshape(1, -1),
      cls_W, cls_b.reshape(1, -1))


def _ceU(We, a_e, mx):
    ce = (We.reshape(HEAD, HID) * a_e).sum(axis=1)
    u = mx[0, :HEAD] + mx[0, HEAD:] + jnp.maximum(ce, 0.0)
    u = jnp.where(u >= 0.0, u, 0.2 * u)
    return jnp.concatenate([ce, u]).astype(jnp.float32)


def kernel(x, edge_index, edge_weight, pre_W, pre_b, W1, att_src1, att_dst1,
           We1, att_edge1, b1, W2, att_src2, att_dst2, We2, att_edge2, b2,
           lin1_W, lin1_b, lin2_W, lin2_b, cls_W, cls_b):
    pad_e = E_PAD - E
    ei_p = jnp.concatenate(
        [edge_index, jnp.full((2, pad_e), N, dtype=edge_index.dtype)], axis=1)
    ew_p = jnp.concatenate(
        [edge_weight[:, 0], jnp.zeros((pad_e,), jnp.float32)])

    x_p = jnp.concatenate(
        [x.reshape(N, 1), jnp.zeros((N_PAD - N, 1), jnp.float32)], axis=0)

    *qs, alo, ahi, mx1 = _p1(
        x_p, pre_W.reshape(1, RD), pre_b.reshape(1, RD), W1,
        att_src1.reshape(1, F), att_dst1.reshape(1, F))
    ceU1 = _ceU(We1, att_edge1, mx1)
    outs1 = _gat_sc(ei_p, ew_p, alo, ahi,
                    jnp.stack(qs).reshape(2, NQ, N_PAD, 2 * HHEAD), ceU1)

    *qs, alo, ahi, mx2 = _mid(
        outs1, b1, W2, att_src2.reshape(1, F), att_dst2.reshape(1, F))
    ceU2 = _ceU(We2, att_edge2, mx2)
    outs2 = _gat_sc(ei_p, ew_p, alo, ahi,
                    jnp.stack(qs).reshape(2, NQ, N_PAD, 2 * HHEAD), ceU2)

    h2 = _combine(outs2)[:N].reshape(500, POOL, F)
    h = _pool(h2, b2).reshape(B, LIN_IN)
    return _dense_head(h, lin1_W, lin1_b, lin2_W, lin2_b, cls_W, cls_b)
